# baseline (speedup 1.0000x reference)
import functools

import jax
import jax.numpy as jnp
from jax import lax
from jax.experimental import pallas as pl
from jax.experimental.pallas import tpu as pltpu
from jax.experimental.pallas import tpu_sc as plsc

N_NODES = 10000
D = 128
NPAD = 10240
NC = 2
NS = 16
NW = NC * NS
CHUNK = 128
ROWS_PT = NPAD // NS

_mesh = lambda: plsc.VectorSubcoreMesh(core_axis_name="c", subcore_axis_name="s")


def _sc_degrees(src3, dst3):
    nchunk = src3.shape[1]

    @functools.partial(
        pl.kernel,
        out_type=jax.ShapeDtypeStruct((NC, 2, NPAD), jnp.float32),
        mesh=_mesh(),
        scratch_types=[
            pltpu.VMEM((nchunk, CHUNK), jnp.int32),
            pltpu.VMEM((nchunk, CHUNK), jnp.int32),
            pltpu.VMEM((CHUNK,), jnp.float32),
            pltpu.VMEM((CHUNK,), jnp.float32),
            pltpu.VMEM_SHARED((NPAD,), jnp.float32),
            pltpu.VMEM_SHARED((NPAD,), jnp.float32),
        ],
    )
    def k(src_hbm, dst_hbm, out_hbm, si_v, di_v, ones_v, zrow_v, dego_sh, degi_sh):
        c = lax.axis_index("c")
        s = lax.axis_index("s")
        wid = s * NC + c
        for i in range(CHUNK // 16):
            ones_v[pl.ds(i * 16, 16)] = jnp.ones((16,), jnp.float32)
            zrow_v[pl.ds(i * 16, 16)] = jnp.zeros((16,), jnp.float32)
        for k2 in range(ROWS_PT // CHUNK):
            off = s * ROWS_PT + k2 * CHUNK
            pltpu.sync_copy(zrow_v, dego_sh.at[pl.ds(off, CHUNK)])
            pltpu.sync_copy(zrow_v, degi_sh.at[pl.ds(off, CHUNK)])
        pltpu.sync_copy(src_hbm.at[wid], si_v)
        pltpu.sync_copy(dst_hbm.at[wid], di_v)
        plsc.subcore_barrier()

        def body(j, carry):
            pltpu.sync_copy(ones_v, dego_sh.at[si_v.at[j]], add=True)
            pltpu.sync_copy(ones_v, degi_sh.at[di_v.at[j]], add=True)
            return carry

        lax.fori_loop(0, nchunk, body, 0)
        plsc.subcore_barrier()
        for k2 in range(ROWS_PT // CHUNK):
            off = s * ROWS_PT + k2 * CHUNK
            pltpu.sync_copy(dego_sh.at[pl.ds(off, CHUNK)],
                            out_hbm.at[c, 0, pl.ds(off, CHUNK)])
            pltpu.sync_copy(degi_sh.at[pl.ds(off, CHUNK)],
                            out_hbm.at[c, 1, pl.ds(off, CHUNK)])

    return k(src3, dst3)


def _sc_propagate(hs, src3, dst3):
    nchunk = src3.shape[1]

    npass = 2
    assert nchunk % (2 * npass) == 0
    cpp = nchunk // npass

    @functools.partial(
        pl.kernel,
        out_type=jax.ShapeDtypeStruct((NC, NPAD, D), jnp.float32),
        mesh=_mesh(),
        scratch_types=[
            pltpu.VMEM((cpp, CHUNK), jnp.int32),
            pltpu.VMEM((cpp, CHUNK), jnp.int32),
            pltpu.VMEM((CHUNK, D), jnp.float32),
            pltpu.VMEM((CHUNK, D), jnp.float32),
            pltpu.VMEM_SHARED((NPAD, D), jnp.float32),
            pltpu.SemaphoreType.DMA,
            pltpu.SemaphoreType.DMA,
        ],
    )
    def k(hs_hbm, src_hbm, dst_hbm, out_hbm, si_v, di_v, rows0, rows1, acc_sh,
          sem0, sem1):
        c = lax.axis_index("c")
        s = lax.axis_index("s")
        wid = s * NC + c

        def zbody(j, carry):
            for i in range(D // 16):
                rows0[j, pl.ds(i * 16, 16)] = jnp.zeros((16,), jnp.float32)
            return carry

        lax.fori_loop(0, CHUNK, zbody, 0)
        for k2 in range(ROWS_PT // CHUNK):
            off = s * ROWS_PT + k2 * CHUNK
            pltpu.sync_copy(rows0, acc_sh.at[pl.ds(off, CHUNK)])
        plsc.subcore_barrier()

        for p in range(npass):
            pltpu.sync_copy(src_hbm.at[wid, pl.ds(p * cpp, cpp)], si_v)
            pltpu.sync_copy(dst_hbm.at[wid, pl.ds(p * cpp, cpp)], di_v)
            def body(j, carry):
                pltpu.async_copy(hs_hbm.at[si_v.at[j]], rows0, sem0).wait()
                pltpu.sync_copy(rows0, acc_sh.at[di_v.at[j]], add=True)
                return carry

            lax.fori_loop(0, cpp, body, 0)
        plsc.subcore_barrier()
        for k2 in range(ROWS_PT // CHUNK):
            off = s * ROWS_PT + k2 * CHUNK
            pltpu.sync_copy(acc_sh.at[pl.ds(off, CHUNK)],
                            out_hbm.at[c, pl.ds(off, CHUNK)])

    return k(hs, src3, dst3)


def _tc_sfactor(degp):
    def body(degp_ref, sc_ref):
        dego = degp_ref[0, 0, :] + degp_ref[1, 0, :]
        degi = degp_ref[0, 1, :] + degp_ref[1, 1, :]
        sc_ref[0, :] = lax.rsqrt(jnp.maximum(dego, 1.0))
        sc_ref[1, :] = lax.rsqrt(jnp.maximum(degi, 1.0))

    return pl.pallas_call(
        body, out_shape=jax.ShapeDtypeStruct((2, NPAD), jnp.float32))(degp)


BM = 512


def _tc_mm_scale(h, W, scn):
    def body(h_ref, w_ref, scn_ref, o_ref):
        acc = jnp.dot(h_ref[...], w_ref[...], preferred_element_type=jnp.float32)
        o_ref[...] = acc * scn_ref[:, 0:1]

    return pl.pallas_call(
        body,
        grid=(NPAD // BM,),
        in_specs=[
            pl.BlockSpec((BM, D), lambda i: (i, 0)),
            pl.BlockSpec((D, D), lambda i: (0, 0)),
            pl.BlockSpec((BM, 2), lambda i: (i, 0)),
        ],
        out_specs=pl.BlockSpec((BM, D), lambda i: (i, 0)),
        out_shape=jax.ShapeDtypeStruct((NPAD, D), jnp.float32),
    )(h, W, scn)


def _tc_mid(P, scn, b0r, W1):
    def body(p_ref, scn_ref, b_ref, w_ref, o_ref):
        agg = p_ref[0] + p_ref[1]
        t = jnp.maximum(agg * scn_ref[:, 1:2] + b_ref[...], 0.0)
        o_ref[...] = jnp.dot(
            t, w_ref[...], preferred_element_type=jnp.float32) * scn_ref[:, 0:1]

    return pl.pallas_call(
        body,
        grid=(NPAD // BM,),
        in_specs=[
            pl.BlockSpec((NC, BM, D), lambda i: (0, i, 0)),
            pl.BlockSpec((BM, 2), lambda i: (i, 0)),
            pl.BlockSpec((1, D), lambda i: (0, 0)),
            pl.BlockSpec((D, D), lambda i: (0, 0)),
        ],
        out_specs=pl.BlockSpec((BM, D), lambda i: (i, 0)),
        out_shape=jax.ShapeDtypeStruct((NPAD, D), jnp.float32),
    )(P, scn, b0r, W1)


def _tc_final(P, scn, b1r):
    def body(p_ref, scn_ref, b_ref, o_ref):
        o_ref[...] = (p_ref[0] + p_ref[1]) * scn_ref[:, 1:2] + b_ref[...]

    return pl.pallas_call(
        body,
        grid=(NPAD // BM,),
        in_specs=[
            pl.BlockSpec((NC, BM, D), lambda i: (0, i, 0)),
            pl.BlockSpec((BM, 2), lambda i: (i, 0)),
            pl.BlockSpec((1, D), lambda i: (0, 0)),
        ],
        out_specs=pl.BlockSpec((BM, D), lambda i: (i, 0)),
        out_shape=jax.ShapeDtypeStruct((NPAD, D), jnp.float32),
    )(P, scn, b1r)


def kernel(x, edge_index, W0, b0, W1, b1):
    n, E = x.shape[0], edge_index.shape[1]
    nchunk = -(-E // (NW * CHUNK))
    nchunk += nchunk % 2
    epad = NW * nchunk * CHUNK - E
    fill = jnp.full((epad,), N_NODES, jnp.int32)
    src3 = jnp.concatenate([edge_index[0], fill]).reshape(NW, nchunk, CHUNK)
    dst3 = jnp.concatenate([edge_index[1], fill]).reshape(NW, nchunk, CHUNK)
    xp = jnp.pad(x, ((0, NPAD - n), (0, 0)))

    degp = _sc_degrees(src3, dst3)
    scn = _tc_sfactor(degp).T
    b0r = b0.reshape(1, D)
    b1r = b1.reshape(1, D)

    hs0 = _tc_mm_scale(xp, W0, scn)
    P0 = _sc_propagate(hs0, src3, dst3)
    hs1 = _tc_mid(P0, scn, b0r, W1)
    P1 = _sc_propagate(hs1, src3, dst3)
    outp = _tc_final(P1, scn, b1r)
    return outp[:n]


# speedup vs baseline: 2.4177x; 2.4177x over previous
import functools

import jax
import jax.numpy as jnp
from jax import lax
from jax.experimental import pallas as pl
from jax.experimental.pallas import tpu as pltpu
from jax.experimental.pallas import tpu_sc as plsc

N_NODES = 10000
D = 128
NPAD = 10240
NC = 2
NS = 16
NW = NC * NS
CHUNK = 128
ROWS_PT = NPAD // NS

_mesh = lambda: plsc.VectorSubcoreMesh(core_axis_name="c", subcore_axis_name="s")


def _sc_degrees(src3, dst3):
    nchunk = src3.shape[1]

    @functools.partial(
        pl.kernel,
        out_type=jax.ShapeDtypeStruct((NC, 2, NPAD), jnp.float32),
        mesh=_mesh(),
        scratch_types=[
            pltpu.VMEM((nchunk, CHUNK), jnp.int32),
            pltpu.VMEM((nchunk, CHUNK), jnp.int32),
            pltpu.VMEM((CHUNK,), jnp.float32),
            pltpu.VMEM((CHUNK,), jnp.float32),
            pltpu.VMEM_SHARED((NPAD,), jnp.float32),
            pltpu.VMEM_SHARED((NPAD,), jnp.float32),
        ],
    )
    def k(src_hbm, dst_hbm, out_hbm, si_v, di_v, ones_v, zrow_v, dego_sh, degi_sh):
        c = lax.axis_index("c")
        s = lax.axis_index("s")
        wid = s * NC + c
        for i in range(CHUNK // 16):
            ones_v[pl.ds(i * 16, 16)] = jnp.ones((16,), jnp.float32)
            zrow_v[pl.ds(i * 16, 16)] = jnp.zeros((16,), jnp.float32)
        for k2 in range(ROWS_PT // CHUNK):
            off = s * ROWS_PT + k2 * CHUNK
            pltpu.sync_copy(zrow_v, dego_sh.at[pl.ds(off, CHUNK)])
            pltpu.sync_copy(zrow_v, degi_sh.at[pl.ds(off, CHUNK)])
        pltpu.sync_copy(src_hbm.at[wid], si_v)
        pltpu.sync_copy(dst_hbm.at[wid], di_v)
        plsc.subcore_barrier()

        def body(j, carry):
            pltpu.sync_copy(ones_v, dego_sh.at[si_v.at[j]], add=True)
            pltpu.sync_copy(ones_v, degi_sh.at[di_v.at[j]], add=True)
            return carry

        lax.fori_loop(0, nchunk, body, 0)
        plsc.subcore_barrier()
        for k2 in range(ROWS_PT // CHUNK):
            off = s * ROWS_PT + k2 * CHUNK
            pltpu.sync_copy(dego_sh.at[pl.ds(off, CHUNK)],
                            out_hbm.at[c, 0, pl.ds(off, CHUNK)])
            pltpu.sync_copy(degi_sh.at[pl.ds(off, CHUNK)],
                            out_hbm.at[c, 1, pl.ds(off, CHUNK)])

    return k(src3, dst3)


def _sc_propagate(hs, src3, dst3):
    nchunk = src3.shape[1]

    npass = 2
    assert nchunk % (2 * npass) == 0
    cpp = nchunk // npass

    @functools.partial(
        pl.kernel,
        out_type=jax.ShapeDtypeStruct((NC, NPAD, D), jnp.float32),
        mesh=_mesh(),
        scratch_types=[
            pltpu.VMEM((cpp, CHUNK), jnp.int32),
            pltpu.VMEM((cpp, CHUNK), jnp.int32),
            pltpu.VMEM((CHUNK, D), jnp.float32),
            pltpu.VMEM((CHUNK, D), jnp.float32),
            pltpu.VMEM_SHARED((NPAD, D), jnp.float32),
            pltpu.SemaphoreType.DMA,
            pltpu.SemaphoreType.DMA,
        ],
    )
    def k(hs_hbm, src_hbm, dst_hbm, out_hbm, si_v, di_v, rows0, rows1, acc_sh,
          sem0, sem1):
        c = lax.axis_index("c")
        s = lax.axis_index("s")
        wid = s * NC + c

        def zbody(j, carry):
            for i in range(D // 16):
                rows0[j, pl.ds(i * 16, 16)] = jnp.zeros((16,), jnp.float32)
            return carry

        lax.fori_loop(0, CHUNK, zbody, 0)
        for k2 in range(ROWS_PT // CHUNK):
            off = s * ROWS_PT + k2 * CHUNK
            pltpu.sync_copy(rows0, acc_sh.at[pl.ds(off, CHUNK)])
        plsc.subcore_barrier()

        for p in range(npass):
            pltpu.sync_copy(src_hbm.at[wid, pl.ds(p * cpp, cpp)], si_v)
            pltpu.sync_copy(dst_hbm.at[wid, pl.ds(p * cpp, cpp)], di_v)
            def body(j, carry):
                pltpu.async_copy(hs_hbm.at[si_v.at[j]], rows0, sem0).wait()
                pltpu.sync_copy(rows0, acc_sh.at[di_v.at[j]], add=True)
                return carry

            lax.fori_loop(0, cpp, body, 0)
        plsc.subcore_barrier()
        for k2 in range(ROWS_PT // CHUNK):
            off = s * ROWS_PT + k2 * CHUNK
            pltpu.sync_copy(acc_sh.at[pl.ds(off, CHUNK)],
                            out_hbm.at[c, pl.ds(off, CHUNK)])

    return k(hs, src3, dst3)


def _tc_sfactor(degp):
    def body(degp_ref, sc_ref):
        dego = degp_ref[0, 0, :] + degp_ref[1, 0, :]
        degi = degp_ref[0, 1, :] + degp_ref[1, 1, :]
        sc_ref[0, :] = lax.rsqrt(jnp.maximum(dego, 1.0))
        sc_ref[1, :] = lax.rsqrt(jnp.maximum(degi, 1.0))

    return pl.pallas_call(
        body, out_shape=jax.ShapeDtypeStruct((2, NPAD), jnp.float32))(degp)


BM = 512


def _tc_mm_scale(h, W, scn):
    def body(h_ref, w_ref, scn_ref, o_ref):
        acc = jnp.dot(h_ref[...], w_ref[...], preferred_element_type=jnp.float32)
        o_ref[...] = acc * scn_ref[:, 0:1]

    return pl.pallas_call(
        body,
        grid=(NPAD // BM,),
        in_specs=[
            pl.BlockSpec((BM, D), lambda i: (i, 0)),
            pl.BlockSpec((D, D), lambda i: (0, 0)),
            pl.BlockSpec((BM, 2), lambda i: (i, 0)),
        ],
        out_specs=pl.BlockSpec((BM, D), lambda i: (i, 0)),
        out_shape=jax.ShapeDtypeStruct((NPAD, D), jnp.float32),
    )(h, W, scn)


def _tc_mid(P, scn, b0r, W1):
    def body(p_ref, scn_ref, b_ref, w_ref, o_ref):
        agg = p_ref[0] + p_ref[1]
        t = jnp.maximum(agg * scn_ref[:, 1:2] + b_ref[...], 0.0)
        o_ref[...] = jnp.dot(
            t, w_ref[...], preferred_element_type=jnp.float32) * scn_ref[:, 0:1]

    return pl.pallas_call(
        body,
        grid=(NPAD // BM,),
        in_specs=[
            pl.BlockSpec((NC, BM, D), lambda i: (0, i, 0)),
            pl.BlockSpec((BM, 2), lambda i: (i, 0)),
            pl.BlockSpec((1, D), lambda i: (0, 0)),
            pl.BlockSpec((D, D), lambda i: (0, 0)),
        ],
        out_specs=pl.BlockSpec((BM, D), lambda i: (i, 0)),
        out_shape=jax.ShapeDtypeStruct((NPAD, D), jnp.float32),
    )(P, scn, b0r, W1)


def _tc_final(P, scn, b1r):
    def body(p_ref, scn_ref, b_ref, o_ref):
        o_ref[...] = (p_ref[0] + p_ref[1]) * scn_ref[:, 1:2] + b_ref[...]

    return pl.pallas_call(
        body,
        grid=(NPAD // BM,),
        in_specs=[
            pl.BlockSpec((NC, BM, D), lambda i: (0, i, 0)),
            pl.BlockSpec((BM, 2), lambda i: (i, 0)),
            pl.BlockSpec((1, D), lambda i: (0, 0)),
        ],
        out_specs=pl.BlockSpec((BM, D), lambda i: (i, 0)),
        out_shape=jax.ShapeDtypeStruct((NPAD, D), jnp.float32),
    )(P, scn, b1r)


def kernel(x, edge_index, W0, b0, W1, b1):
    n, E = x.shape[0], edge_index.shape[1]
    nchunk = -(-E // (NW * CHUNK))
    nchunk += nchunk % 2
    epad = NW * nchunk * CHUNK - E
    fill = N_NODES + jnp.arange(epad, dtype=jnp.int32) % (NPAD - N_NODES)
    src3 = jnp.concatenate([edge_index[0], fill]).reshape(NW, nchunk, CHUNK)
    dst3 = jnp.concatenate([edge_index[1], fill]).reshape(NW, nchunk, CHUNK)
    xp = jnp.pad(x, ((0, NPAD - n), (0, 0)))

    degp = _sc_degrees(src3, dst3)
    scn = _tc_sfactor(degp).T
    b0r = b0.reshape(1, D)
    b1r = b1.reshape(1, D)

    hs0 = _tc_mm_scale(xp, W0, scn)
    P0 = _sc_propagate(hs0, src3, dst3)
    hs1 = _tc_mid(P0, scn, b0r, W1)
    P1 = _sc_propagate(hs1, src3, dst3)
    outp = _tc_final(P1, scn, b1r)
    return outp[:n]


# speedup vs baseline: 3.0124x; 1.2460x over previous
import functools

import jax
import jax.numpy as jnp
from jax import lax
from jax.experimental import pallas as pl
from jax.experimental.pallas import tpu as pltpu
from jax.experimental.pallas import tpu_sc as plsc

N_NODES = 10000
D = 128
NPAD = 10240
NC = 2
NS = 16
NW = NC * NS
CHUNK = 128
ROWS_PT = NPAD // NS

_mesh = lambda: plsc.VectorSubcoreMesh(core_axis_name="c", subcore_axis_name="s")


def _sc_degrees(src3, dst3):
    nchunk = src3.shape[1]

    @functools.partial(
        pl.kernel,
        out_type=jax.ShapeDtypeStruct((NC, 2, NPAD), jnp.float32),
        mesh=_mesh(),
        scratch_types=[
            pltpu.VMEM((nchunk, CHUNK), jnp.int32),
            pltpu.VMEM((nchunk, CHUNK), jnp.int32),
            pltpu.VMEM((CHUNK,), jnp.float32),
            pltpu.VMEM((CHUNK,), jnp.float32),
            pltpu.VMEM_SHARED((NPAD,), jnp.float32),
            pltpu.VMEM_SHARED((NPAD,), jnp.float32),
        ],
    )
    def k(src_hbm, dst_hbm, out_hbm, si_v, di_v, ones_v, zrow_v, dego_sh, degi_sh):
        c = lax.axis_index("c")
        s = lax.axis_index("s")
        wid = s * NC + c
        for i in range(CHUNK // 16):
            ones_v[pl.ds(i * 16, 16)] = jnp.ones((16,), jnp.float32)
            zrow_v[pl.ds(i * 16, 16)] = jnp.zeros((16,), jnp.float32)
        for k2 in range(ROWS_PT // CHUNK):
            off = s * ROWS_PT + k2 * CHUNK
            pltpu.sync_copy(zrow_v, dego_sh.at[pl.ds(off, CHUNK)])
            pltpu.sync_copy(zrow_v, degi_sh.at[pl.ds(off, CHUNK)])
        pltpu.sync_copy(src_hbm.at[wid], si_v)
        pltpu.sync_copy(dst_hbm.at[wid], di_v)
        plsc.subcore_barrier()

        def body(j, carry):
            pltpu.sync_copy(ones_v, dego_sh.at[si_v.at[j]], add=True)
            pltpu.sync_copy(ones_v, degi_sh.at[di_v.at[j]], add=True)
            return carry

        lax.fori_loop(0, nchunk, body, 0)
        plsc.subcore_barrier()
        for k2 in range(ROWS_PT // CHUNK):
            off = s * ROWS_PT + k2 * CHUNK
            pltpu.sync_copy(dego_sh.at[pl.ds(off, CHUNK)],
                            out_hbm.at[c, 0, pl.ds(off, CHUNK)])
            pltpu.sync_copy(degi_sh.at[pl.ds(off, CHUNK)],
                            out_hbm.at[c, 1, pl.ds(off, CHUNK)])

    return k(src3, dst3)


def _sc_propagate(hs, src3, dst3):
    nchunk = src3.shape[1]

    npass = 2
    assert nchunk % (2 * npass) == 0
    cpp = nchunk // npass

    @functools.partial(
        pl.kernel,
        out_type=jax.ShapeDtypeStruct((NC, NPAD, D), jnp.float32),
        mesh=_mesh(),
        scratch_types=[
            pltpu.VMEM((cpp, CHUNK), jnp.int32),
            pltpu.VMEM((cpp, CHUNK), jnp.int32),
            pltpu.VMEM((CHUNK, D), jnp.float32),
            pltpu.VMEM((CHUNK, D), jnp.float32),
            pltpu.VMEM_SHARED((NPAD, D), jnp.float32),
            pltpu.SemaphoreType.DMA,
            pltpu.SemaphoreType.DMA,
        ],
    )
    def k(hs_hbm, src_hbm, dst_hbm, out_hbm, si_v, di_v, rows0, rows1, acc_sh,
          sem0, sem1):
        c = lax.axis_index("c")
        s = lax.axis_index("s")
        wid = s * NC + c

        def zbody(j, carry):
            for i in range(D // 16):
                rows0[j, pl.ds(i * 16, 16)] = jnp.zeros((16,), jnp.float32)
            return carry

        lax.fori_loop(0, CHUNK, zbody, 0)
        for k2 in range(ROWS_PT // CHUNK):
            off = s * ROWS_PT + k2 * CHUNK
            pltpu.sync_copy(rows0, acc_sh.at[pl.ds(off, CHUNK)])
        plsc.subcore_barrier()

        for p in range(npass):
            pltpu.sync_copy(src_hbm.at[wid, pl.ds(p * cpp, cpp)], si_v)
            pltpu.sync_copy(dst_hbm.at[wid, pl.ds(p * cpp, cpp)], di_v)
            pltpu.async_copy(hs_hbm.at[si_v.at[0]], rows0, sem0)

            def body(j2, carry):
                j = 2 * j2
                pltpu.make_async_copy(hs_hbm.at[si_v.at[j]], rows0, sem0).wait()
                pltpu.async_copy(hs_hbm.at[si_v.at[j + 1]], rows1, sem1)
                pltpu.sync_copy(rows0, acc_sh.at[di_v.at[j]], add=True)
                pltpu.make_async_copy(
                    hs_hbm.at[si_v.at[j + 1]], rows1, sem1).wait()
                pltpu.async_copy(hs_hbm.at[si_v.at[j + 2]], rows0, sem0)
                pltpu.sync_copy(rows1, acc_sh.at[di_v.at[j + 1]], add=True)
                return carry

            lax.fori_loop(0, (cpp - 2) // 2, body, 0)
            pltpu.make_async_copy(hs_hbm.at[si_v.at[cpp - 2]], rows0,
                                  sem0).wait()
            pltpu.async_copy(hs_hbm.at[si_v.at[cpp - 1]], rows1, sem1)
            pltpu.sync_copy(rows0, acc_sh.at[di_v.at[cpp - 2]], add=True)
            pltpu.make_async_copy(hs_hbm.at[si_v.at[cpp - 1]], rows1,
                                  sem1).wait()
            pltpu.sync_copy(rows1, acc_sh.at[di_v.at[cpp - 1]], add=True)
        plsc.subcore_barrier()
        for k2 in range(ROWS_PT // CHUNK):
            off = s * ROWS_PT + k2 * CHUNK
            pltpu.sync_copy(acc_sh.at[pl.ds(off, CHUNK)],
                            out_hbm.at[c, pl.ds(off, CHUNK)])

    return k(hs, src3, dst3)


def _tc_sfactor(degp):
    def body(degp_ref, sc_ref):
        dego = degp_ref[0, 0, :] + degp_ref[1, 0, :]
        degi = degp_ref[0, 1, :] + degp_ref[1, 1, :]
        sc_ref[0, :] = lax.rsqrt(jnp.maximum(dego, 1.0))
        sc_ref[1, :] = lax.rsqrt(jnp.maximum(degi, 1.0))

    return pl.pallas_call(
        body, out_shape=jax.ShapeDtypeStruct((2, NPAD), jnp.float32))(degp)


BM = 512


def _tc_mm_scale(h, W, scn):
    def body(h_ref, w_ref, scn_ref, o_ref):
        acc = jnp.dot(h_ref[...], w_ref[...], preferred_element_type=jnp.float32)
        o_ref[...] = acc * scn_ref[:, 0:1]

    return pl.pallas_call(
        body,
        grid=(NPAD // BM,),
        in_specs=[
            pl.BlockSpec((BM, D), lambda i: (i, 0)),
            pl.BlockSpec((D, D), lambda i: (0, 0)),
            pl.BlockSpec((BM, 2), lambda i: (i, 0)),
        ],
        out_specs=pl.BlockSpec((BM, D), lambda i: (i, 0)),
        out_shape=jax.ShapeDtypeStruct((NPAD, D), jnp.float32),
    )(h, W, scn)


def _tc_mid(P, scn, b0r, W1):
    def body(p_ref, scn_ref, b_ref, w_ref, o_ref):
        agg = p_ref[0] + p_ref[1]
        t = jnp.maximum(agg * scn_ref[:, 1:2] + b_ref[...], 0.0)
        o_ref[...] = jnp.dot(
            t, w_ref[...], preferred_element_type=jnp.float32) * scn_ref[:, 0:1]

    return pl.pallas_call(
        body,
        grid=(NPAD // BM,),
        in_specs=[
            pl.BlockSpec((NC, BM, D), lambda i: (0, i, 0)),
            pl.BlockSpec((BM, 2), lambda i: (i, 0)),
            pl.BlockSpec((1, D), lambda i: (0, 0)),
            pl.BlockSpec((D, D), lambda i: (0, 0)),
        ],
        out_specs=pl.BlockSpec((BM, D), lambda i: (i, 0)),
        out_shape=jax.ShapeDtypeStruct((NPAD, D), jnp.float32),
    )(P, scn, b0r, W1)


def _tc_final(P, scn, b1r):
    def body(p_ref, scn_ref, b_ref, o_ref):
        o_ref[...] = (p_ref[0] + p_ref[1]) * scn_ref[:, 1:2] + b_ref[...]

    return pl.pallas_call(
        body,
        grid=(NPAD // BM,),
        in_specs=[
            pl.BlockSpec((NC, BM, D), lambda i: (0, i, 0)),
            pl.BlockSpec((BM, 2), lambda i: (i, 0)),
            pl.BlockSpec((1, D), lambda i: (0, 0)),
        ],
        out_specs=pl.BlockSpec((BM, D), lambda i: (i, 0)),
        out_shape=jax.ShapeDtypeStruct((NPAD, D), jnp.float32),
    )(P, scn, b1r)


def kernel(x, edge_index, W0, b0, W1, b1):
    n, E = x.shape[0], edge_index.shape[1]
    nchunk = -(-E // (NW * CHUNK))
    nchunk += nchunk % 2
    epad = NW * nchunk * CHUNK - E
    fill = N_NODES + jnp.arange(epad, dtype=jnp.int32) % (NPAD - N_NODES)
    src3 = jnp.concatenate([edge_index[0], fill]).reshape(NW, nchunk, CHUNK)
    dst3 = jnp.concatenate([edge_index[1], fill]).reshape(NW, nchunk, CHUNK)
    xp = jnp.pad(x, ((0, NPAD - n), (0, 0)))

    degp = _sc_degrees(src3, dst3)
    scn = _tc_sfactor(degp).T
    b0r = b0.reshape(1, D)
    b1r = b1.reshape(1, D)

    hs0 = _tc_mm_scale(xp, W0, scn)
    P0 = _sc_propagate(hs0, src3, dst3)
    hs1 = _tc_mid(P0, scn, b0r, W1)
    P1 = _sc_propagate(hs1, src3, dst3)
    outp = _tc_final(P1, scn, b1r)
    return outp[:n]


# speedup vs baseline: 3.0313x; 1.0063x over previous
import functools

import jax
import jax.numpy as jnp
from jax import lax
from jax.experimental import pallas as pl
from jax.experimental.pallas import tpu as pltpu
from jax.experimental.pallas import tpu_sc as plsc

N_NODES = 10000
D = 128
NPAD = 10240
NC = 2
NS = 16
NW = NC * NS
CHUNK = 128
ROWS_PT = NPAD // NS

_mesh = lambda: plsc.VectorSubcoreMesh(core_axis_name="c", subcore_axis_name="s")


def _sc_degrees(src3, dst3):
    nchunk = src3.shape[1]

    @functools.partial(
        pl.kernel,
        out_type=jax.ShapeDtypeStruct((NC, 2, NPAD), jnp.float32),
        mesh=_mesh(),
        scratch_types=[
            pltpu.VMEM((nchunk, CHUNK), jnp.int32),
            pltpu.VMEM((nchunk, CHUNK), jnp.int32),
            pltpu.VMEM((CHUNK,), jnp.float32),
            pltpu.VMEM((CHUNK,), jnp.float32),
            pltpu.VMEM_SHARED((NPAD,), jnp.float32),
            pltpu.VMEM_SHARED((NPAD,), jnp.float32),
        ],
    )
    def k(src_hbm, dst_hbm, out_hbm, si_v, di_v, ones_v, zrow_v, dego_sh, degi_sh):
        c = lax.axis_index("c")
        s = lax.axis_index("s")
        wid = s * NC + c
        for i in range(CHUNK // 16):
            ones_v[pl.ds(i * 16, 16)] = jnp.ones((16,), jnp.float32)
            zrow_v[pl.ds(i * 16, 16)] = jnp.zeros((16,), jnp.float32)
        for k2 in range(ROWS_PT // CHUNK):
            off = s * ROWS_PT + k2 * CHUNK
            pltpu.sync_copy(zrow_v, dego_sh.at[pl.ds(off, CHUNK)])
            pltpu.sync_copy(zrow_v, degi_sh.at[pl.ds(off, CHUNK)])
        pltpu.sync_copy(src_hbm.at[wid], si_v)
        pltpu.sync_copy(dst_hbm.at[wid], di_v)
        plsc.subcore_barrier()

        def body(j, carry):
            pltpu.sync_copy(ones_v, dego_sh.at[si_v.at[j]], add=True)
            pltpu.sync_copy(ones_v, degi_sh.at[di_v.at[j]], add=True)
            return carry

        lax.fori_loop(0, nchunk, body, 0)
        plsc.subcore_barrier()
        for k2 in range(ROWS_PT // CHUNK):
            off = s * ROWS_PT + k2 * CHUNK
            pltpu.sync_copy(dego_sh.at[pl.ds(off, CHUNK)],
                            out_hbm.at[c, 0, pl.ds(off, CHUNK)])
            pltpu.sync_copy(degi_sh.at[pl.ds(off, CHUNK)],
                            out_hbm.at[c, 1, pl.ds(off, CHUNK)])

    return k(src3, dst3)


def _sc_propagate(hs, src3, dst3):
    nchunk = src3.shape[1]

    npass = 2
    assert nchunk % (2 * npass) == 0
    cpp = nchunk // npass

    @functools.partial(
        pl.kernel,
        out_type=jax.ShapeDtypeStruct((NC, NPAD, D), jnp.float32),
        mesh=_mesh(),
        scratch_types=[
            pltpu.VMEM((cpp, CHUNK), jnp.int32),
            pltpu.VMEM((cpp, CHUNK), jnp.int32),
            pltpu.VMEM((CHUNK, D), jnp.float32),
            pltpu.VMEM((CHUNK, D), jnp.float32),
            pltpu.VMEM_SHARED((NPAD, D), jnp.float32),
            pltpu.SemaphoreType.DMA,
            pltpu.SemaphoreType.DMA,
        ],
    )
    def k(hs_hbm, src_hbm, dst_hbm, out_hbm, si_v, di_v, rows0, rows1, acc_sh,
          sem0, sem1):
        c = lax.axis_index("c")
        s = lax.axis_index("s")
        wid = s * NC + c

        def zbody(j, carry):
            for i in range(D // 16):
                rows0[j, pl.ds(i * 16, 16)] = jnp.zeros((16,), jnp.float32)
            return carry

        lax.fori_loop(0, CHUNK, zbody, 0)
        for k2 in range(ROWS_PT // CHUNK):
            off = s * ROWS_PT + k2 * CHUNK
            pltpu.sync_copy(rows0, acc_sh.at[pl.ds(off, CHUNK)])
        plsc.subcore_barrier()

        for p in range(npass):
            pltpu.sync_copy(src_hbm.at[wid, pl.ds(p * cpp, cpp)], si_v)
            pltpu.sync_copy(dst_hbm.at[wid, pl.ds(p * cpp, cpp)], di_v)
            pltpu.async_copy(hs_hbm.at[si_v.at[0]], rows0, sem0)

            def body(j2, carry):
                j = 2 * j2
                pltpu.make_async_copy(hs_hbm.at[si_v.at[j]], rows0, sem0).wait()
                pltpu.async_copy(hs_hbm.at[si_v.at[j + 1]], rows1, sem1)
                pltpu.sync_copy(rows0, acc_sh.at[di_v.at[j]], add=True)
                pltpu.make_async_copy(
                    hs_hbm.at[si_v.at[j + 1]], rows1, sem1).wait()
                pltpu.async_copy(hs_hbm.at[si_v.at[j + 2]], rows0, sem0)
                pltpu.sync_copy(rows1, acc_sh.at[di_v.at[j + 1]], add=True)
                return carry

            lax.fori_loop(0, (cpp - 2) // 2, body, 0)
            pltpu.make_async_copy(hs_hbm.at[si_v.at[cpp - 2]], rows0,
                                  sem0).wait()
            pltpu.async_copy(hs_hbm.at[si_v.at[cpp - 1]], rows1, sem1)
            pltpu.sync_copy(rows0, acc_sh.at[di_v.at[cpp - 2]], add=True)
            pltpu.make_async_copy(hs_hbm.at[si_v.at[cpp - 1]], rows1,
                                  sem1).wait()
            pltpu.sync_copy(rows1, acc_sh.at[di_v.at[cpp - 1]], add=True)
        plsc.subcore_barrier()
        for k2 in range(ROWS_PT // CHUNK):
            off = s * ROWS_PT + k2 * CHUNK
            pltpu.sync_copy(acc_sh.at[pl.ds(off, CHUNK)],
                            out_hbm.at[c, pl.ds(off, CHUNK)])

    return k(hs, src3, dst3)


def _tc_sfactor(degp):
    def body(degp_ref, sc_ref):
        dego = degp_ref[0, 0, :] + degp_ref[1, 0, :]
        degi = degp_ref[0, 1, :] + degp_ref[1, 1, :]
        sc_ref[0, :] = lax.rsqrt(jnp.maximum(dego, 1.0))
        sc_ref[1, :] = lax.rsqrt(jnp.maximum(degi, 1.0))

    return pl.pallas_call(
        body, out_shape=jax.ShapeDtypeStruct((2, NPAD), jnp.float32))(degp)


BM = 512


def _tc_mm0(h, W):
    def body(h_ref, w_ref, o_ref):
        o_ref[...] = jnp.dot(h_ref[...], w_ref[...],
                             preferred_element_type=jnp.float32)

    return pl.pallas_call(
        body,
        grid=(NPAD // BM,),
        in_specs=[
            pl.BlockSpec((BM, D), lambda i: (i, 0)),
            pl.BlockSpec((D, D), lambda i: (0, 0)),
        ],
        out_specs=pl.BlockSpec((BM, D), lambda i: (i, 0)),
        out_shape=jax.ShapeDtypeStruct((NPAD, D), jnp.float32),
    )(h, W)


def _tc_scale0(h, scn):
    def body(h_ref, scn_ref, o_ref):
        o_ref[...] = h_ref[...] * scn_ref[:, 0:1]

    return pl.pallas_call(
        body,
        grid=(NPAD // BM,),
        in_specs=[
            pl.BlockSpec((BM, D), lambda i: (i, 0)),
            pl.BlockSpec((BM, 2), lambda i: (i, 0)),
        ],
        out_specs=pl.BlockSpec((BM, D), lambda i: (i, 0)),
        out_shape=jax.ShapeDtypeStruct((NPAD, D), jnp.float32),
    )(h, scn)


def _tc_mid(P, scn, b0r, W1):
    def body(p_ref, scn_ref, b_ref, w_ref, o_ref):
        agg = p_ref[0] + p_ref[1]
        t = jnp.maximum(agg * scn_ref[:, 1:2] + b_ref[...], 0.0)
        o_ref[...] = jnp.dot(
            t, w_ref[...], preferred_element_type=jnp.float32) * scn_ref[:, 0:1]

    return pl.pallas_call(
        body,
        grid=(NPAD // BM,),
        in_specs=[
            pl.BlockSpec((NC, BM, D), lambda i: (0, i, 0)),
            pl.BlockSpec((BM, 2), lambda i: (i, 0)),
            pl.BlockSpec((1, D), lambda i: (0, 0)),
            pl.BlockSpec((D, D), lambda i: (0, 0)),
        ],
        out_specs=pl.BlockSpec((BM, D), lambda i: (i, 0)),
        out_shape=jax.ShapeDtypeStruct((NPAD, D), jnp.float32),
    )(P, scn, b0r, W1)


BMF = 400


def _tc_final(P, scn, b1r):
    def body(p_ref, scn_ref, b_ref, o_ref):
        o_ref[...] = (p_ref[0] + p_ref[1]) * scn_ref[:, 1:2] + b_ref[...]

    return pl.pallas_call(
        body,
        grid=(N_NODES // BMF,),
        in_specs=[
            pl.BlockSpec((NC, BMF, D), lambda i: (0, i, 0)),
            pl.BlockSpec((BMF, 2), lambda i: (i, 0)),
            pl.BlockSpec((1, D), lambda i: (0, 0)),
        ],
        out_specs=pl.BlockSpec((BMF, D), lambda i: (i, 0)),
        out_shape=jax.ShapeDtypeStruct((N_NODES, D), jnp.float32),
    )(P, scn, b1r)


def kernel(x, edge_index, W0, b0, W1, b1):
    n, E = x.shape[0], edge_index.shape[1]
    nchunk = -(-E // (NW * CHUNK))
    nchunk += nchunk % 2
    epad = NW * nchunk * CHUNK - E
    fill = N_NODES + jnp.arange(epad, dtype=jnp.int32) % (NPAD - N_NODES)
    src3 = jnp.concatenate([edge_index[0], fill]).reshape(NW, nchunk, CHUNK)
    dst3 = jnp.concatenate([edge_index[1], fill]).reshape(NW, nchunk, CHUNK)
    xp = jnp.pad(x, ((0, NPAD - n), (0, 0)))

    degp = _sc_degrees(src3, dst3)
    h0 = _tc_mm0(xp, W0)
    scn = _tc_sfactor(degp).T
    b0r = b0.reshape(1, D)
    b1r = b1.reshape(1, D)

    hs0 = _tc_scale0(h0, scn)
    P0 = _sc_propagate(hs0, src3, dst3)
    hs1 = _tc_mid(P0, scn, b0r, W1)
    P1 = _sc_propagate(hs1, src3, dst3)
    return _tc_final(P1, scn, b1r)


# speedup vs baseline: 3.1981x; 1.0550x over previous
import functools

import jax
import jax.numpy as jnp
from jax import lax
from jax.experimental import pallas as pl
from jax.experimental.pallas import tpu as pltpu
from jax.experimental.pallas import tpu_sc as plsc

N_NODES = 10000
D = 128
NPAD = 10240
NC = 2
NS = 16
NW = NC * NS
CHUNK = 128
ROWS_PT = NPAD // NS

_mesh = lambda: plsc.VectorSubcoreMesh(core_axis_name="c", subcore_axis_name="s")


def _sc_degrees(src3, dst3):
    nchunk = src3.shape[1]

    @functools.partial(
        pl.kernel,
        out_type=jax.ShapeDtypeStruct((NC, 2, NPAD), jnp.float32),
        mesh=_mesh(),
        scratch_types=[
            pltpu.VMEM((nchunk, CHUNK), jnp.int32),
            pltpu.VMEM((nchunk, CHUNK), jnp.int32),
            pltpu.VMEM((CHUNK,), jnp.float32),
            pltpu.VMEM((CHUNK,), jnp.float32),
            pltpu.VMEM_SHARED((NPAD,), jnp.float32),
            pltpu.VMEM_SHARED((NPAD,), jnp.float32),
        ],
    )
    def k(src_hbm, dst_hbm, out_hbm, si_v, di_v, ones_v, zrow_v, dego_sh, degi_sh):
        c = lax.axis_index("c")
        s = lax.axis_index("s")
        wid = s * NC + c
        for i in range(CHUNK // 16):
            ones_v[pl.ds(i * 16, 16)] = jnp.ones((16,), jnp.float32)
            zrow_v[pl.ds(i * 16, 16)] = jnp.zeros((16,), jnp.float32)
        for k2 in range(ROWS_PT // CHUNK):
            off = s * ROWS_PT + k2 * CHUNK
            pltpu.sync_copy(zrow_v, dego_sh.at[pl.ds(off, CHUNK)])
            pltpu.sync_copy(zrow_v, degi_sh.at[pl.ds(off, CHUNK)])
        pltpu.sync_copy(src_hbm.at[wid], si_v)
        pltpu.sync_copy(dst_hbm.at[wid], di_v)
        plsc.subcore_barrier()

        def body(j, carry):
            pltpu.sync_copy(ones_v, dego_sh.at[si_v.at[j]], add=True)
            pltpu.sync_copy(ones_v, degi_sh.at[di_v.at[j]], add=True)
            return carry

        lax.fori_loop(0, nchunk, body, 0)
        plsc.subcore_barrier()
        for k2 in range(ROWS_PT // CHUNK):
            off = s * ROWS_PT + k2 * CHUNK
            pltpu.sync_copy(dego_sh.at[pl.ds(off, CHUNK)],
                            out_hbm.at[c, 0, pl.ds(off, CHUNK)])
            pltpu.sync_copy(degi_sh.at[pl.ds(off, CHUNK)],
                            out_hbm.at[c, 1, pl.ds(off, CHUNK)])

    return k(src3, dst3)


def _sc_propagate(hs, src3, dst3):
    nchunk = src3.shape[1]

    npass = 2
    assert nchunk % (2 * npass) == 0
    cpp = nchunk // npass

    @functools.partial(
        pl.kernel,
        out_type=jax.ShapeDtypeStruct((NC, NPAD, D), jnp.float32),
        mesh=_mesh(),
        scratch_types=[
            pltpu.VMEM((cpp, CHUNK), jnp.int32),
            pltpu.VMEM((cpp, CHUNK), jnp.int32),
            pltpu.VMEM((CHUNK, D), jnp.float32),
            pltpu.VMEM((CHUNK, D), jnp.float32),
            pltpu.VMEM_SHARED((NPAD, D), jnp.float32),
            pltpu.SemaphoreType.DMA,
            pltpu.SemaphoreType.DMA,
        ],
    )
    def k(hs_hbm, src_hbm, dst_hbm, out_hbm, si_v, di_v, rows0, rows1, acc_sh,
          sem0, sem1):
        c = lax.axis_index("c")
        s = lax.axis_index("s")
        wid = s * NC + c

        def zbody(j, carry):
            for i in range(D // 16):
                rows0[j, pl.ds(i * 16, 16)] = jnp.zeros((16,), jnp.float32)
            return carry

        lax.fori_loop(0, CHUNK, zbody, 0)
        for k2 in range(ROWS_PT // CHUNK):
            off = s * ROWS_PT + k2 * CHUNK
            pltpu.sync_copy(rows0, acc_sh.at[pl.ds(off, CHUNK)])
        plsc.subcore_barrier()

        for p in range(npass):
            pltpu.sync_copy(src_hbm.at[wid, pl.ds(p * cpp, cpp)], si_v)
            pltpu.sync_copy(dst_hbm.at[wid, pl.ds(p * cpp, cpp)], di_v)
            pltpu.async_copy(hs_hbm.at[si_v.at[0]], rows0, sem0)

            def body(j2, carry):
                j = 2 * j2
                pltpu.make_async_copy(hs_hbm.at[si_v.at[j]], rows0, sem0).wait()
                pltpu.async_copy(hs_hbm.at[si_v.at[j + 1]], rows1, sem1)
                pltpu.sync_copy(rows0, acc_sh.at[di_v.at[j]], add=True)
                pltpu.make_async_copy(
                    hs_hbm.at[si_v.at[j + 1]], rows1, sem1).wait()
                pltpu.async_copy(hs_hbm.at[si_v.at[j + 2]], rows0, sem0)
                pltpu.sync_copy(rows1, acc_sh.at[di_v.at[j + 1]], add=True)
                return carry

            lax.fori_loop(0, (cpp - 2) // 2, body, 0)
            pltpu.make_async_copy(hs_hbm.at[si_v.at[cpp - 2]], rows0,
                                  sem0).wait()
            pltpu.async_copy(hs_hbm.at[si_v.at[cpp - 1]], rows1, sem1)
            pltpu.sync_copy(rows0, acc_sh.at[di_v.at[cpp - 2]], add=True)
            pltpu.make_async_copy(hs_hbm.at[si_v.at[cpp - 1]], rows1,
                                  sem1).wait()
            pltpu.sync_copy(rows1, acc_sh.at[di_v.at[cpp - 1]], add=True)
        plsc.subcore_barrier()
        for k2 in range(ROWS_PT // CHUNK):
            off = s * ROWS_PT + k2 * CHUNK
            pltpu.sync_copy(acc_sh.at[pl.ds(off, CHUNK)],
                            out_hbm.at[c, pl.ds(off, CHUNK)])

    return k(hs, src3, dst3)


def _tc_sfactor(degp):
    def body(degp_ref, sc_ref):
        dego = degp_ref[0, 0, :] + degp_ref[1, 0, :]
        degi = degp_ref[0, 1, :] + degp_ref[1, 1, :]
        sc_ref[0, :] = lax.rsqrt(jnp.maximum(dego, 1.0))
        sc_ref[1, :] = lax.rsqrt(jnp.maximum(degi, 1.0))

    return pl.pallas_call(
        body, out_shape=jax.ShapeDtypeStruct((2, NPAD), jnp.float32))(degp)


BM = 1280


def _tc_mm0(h, W):
    def body(h_ref, w_ref, o_ref):
        o_ref[...] = jnp.dot(h_ref[...], w_ref[...],
                             preferred_element_type=jnp.float32)

    return pl.pallas_call(
        body,
        grid=(NPAD // BM,),
        in_specs=[
            pl.BlockSpec((BM, D), lambda i: (i, 0)),
            pl.BlockSpec((D, D), lambda i: (0, 0)),
        ],
        out_specs=pl.BlockSpec((BM, D), lambda i: (i, 0)),
        out_shape=jax.ShapeDtypeStruct((NPAD, D), jnp.float32),
    )(h, W)


def _tc_scale0(h, scn):
    def body(h_ref, scn_ref, o_ref):
        o_ref[...] = h_ref[...] * scn_ref[:, 0:1]

    return pl.pallas_call(
        body,
        grid=(NPAD // BM,),
        in_specs=[
            pl.BlockSpec((BM, D), lambda i: (i, 0)),
            pl.BlockSpec((BM, 2), lambda i: (i, 0)),
        ],
        out_specs=pl.BlockSpec((BM, D), lambda i: (i, 0)),
        out_shape=jax.ShapeDtypeStruct((NPAD, D), jnp.float32),
    )(h, scn)


def _tc_mid(P, scn, b0r, W1):
    def body(p_ref, scn_ref, b_ref, w_ref, o_ref):
        agg = p_ref[0] + p_ref[1]
        t = jnp.maximum(agg * scn_ref[:, 1:2] + b_ref[...], 0.0)
        o_ref[...] = jnp.dot(
            t, w_ref[...], preferred_element_type=jnp.float32) * scn_ref[:, 0:1]

    return pl.pallas_call(
        body,
        grid=(NPAD // BM,),
        in_specs=[
            pl.BlockSpec((NC, BM, D), lambda i: (0, i, 0)),
            pl.BlockSpec((BM, 2), lambda i: (i, 0)),
            pl.BlockSpec((1, D), lambda i: (0, 0)),
            pl.BlockSpec((D, D), lambda i: (0, 0)),
        ],
        out_specs=pl.BlockSpec((BM, D), lambda i: (i, 0)),
        out_shape=jax.ShapeDtypeStruct((NPAD, D), jnp.float32),
    )(P, scn, b0r, W1)


BMF = 1000


def _tc_final(P, scn, b1r):
    def body(p_ref, scn_ref, b_ref, o_ref):
        o_ref[...] = (p_ref[0] + p_ref[1]) * scn_ref[:, 1:2] + b_ref[...]

    return pl.pallas_call(
        body,
        grid=(N_NODES // BMF,),
        in_specs=[
            pl.BlockSpec((NC, BMF, D), lambda i: (0, i, 0)),
            pl.BlockSpec((BMF, 2), lambda i: (i, 0)),
            pl.BlockSpec((1, D), lambda i: (0, 0)),
        ],
        out_specs=pl.BlockSpec((BMF, D), lambda i: (i, 0)),
        out_shape=jax.ShapeDtypeStruct((N_NODES, D), jnp.float32),
    )(P, scn, b1r)


def kernel(x, edge_index, W0, b0, W1, b1):
    n, E = x.shape[0], edge_index.shape[1]
    nchunk = -(-E // (NW * CHUNK))
    nchunk += nchunk % 2
    epad = NW * nchunk * CHUNK - E
    fill = N_NODES + jnp.arange(epad, dtype=jnp.int32) % (NPAD - N_NODES)
    src3 = jnp.concatenate([edge_index[0], fill]).reshape(NW, nchunk, CHUNK)
    dst3 = jnp.concatenate([edge_index[1], fill]).reshape(NW, nchunk, CHUNK)
    xp = jnp.pad(x, ((0, NPAD - n), (0, 0)))

    degp = _sc_degrees(src3, dst3)
    h0 = _tc_mm0(xp, W0)
    scn = _tc_sfactor(degp).T
    b0r = b0.reshape(1, D)
    b1r = b1.reshape(1, D)

    hs0 = _tc_scale0(h0, scn)
    P0 = _sc_propagate(hs0, src3, dst3)
    hs1 = _tc_mid(P0, scn, b0r, W1)
    P1 = _sc_propagate(hs1, src3, dst3)
    return _tc_final(P1, scn, b1r)


# speedup vs baseline: 3.2410x; 1.0134x over previous
import functools

import jax
import jax.numpy as jnp
from jax import lax
from jax.experimental import pallas as pl
from jax.experimental.pallas import tpu as pltpu
from jax.experimental.pallas import tpu_sc as plsc

N_NODES = 10000
D = 128
NPAD = 10240
NC = 2
NS = 16
NW = NC * NS
CHUNK = 128
ROWS_PT = NPAD // NS

_mesh = lambda: plsc.VectorSubcoreMesh(core_axis_name="c", subcore_axis_name="s")


def _sc_degrees(edges4):
    nchunk = edges4.shape[2]

    @functools.partial(
        pl.kernel,
        out_type=jax.ShapeDtypeStruct((NC, 2, NPAD), jnp.float32),
        mesh=_mesh(),
        scratch_types=[
            pltpu.VMEM((nchunk, CHUNK), jnp.int32),
            pltpu.VMEM((nchunk, CHUNK), jnp.int32),
            pltpu.VMEM((CHUNK,), jnp.float32),
            pltpu.VMEM((CHUNK,), jnp.float32),
            pltpu.VMEM_SHARED((NPAD,), jnp.float32),
            pltpu.VMEM_SHARED((NPAD,), jnp.float32),
        ],
    )
    def k(edges_hbm, out_hbm, si_v, di_v, ones_v, zrow_v, dego_sh, degi_sh):
        c = lax.axis_index("c")
        s = lax.axis_index("s")
        wid = s * NC + c
        for i in range(CHUNK // 16):
            ones_v[pl.ds(i * 16, 16)] = jnp.ones((16,), jnp.float32)
            zrow_v[pl.ds(i * 16, 16)] = jnp.zeros((16,), jnp.float32)
        for k2 in range(ROWS_PT // CHUNK):
            off = s * ROWS_PT + k2 * CHUNK
            pltpu.sync_copy(zrow_v, dego_sh.at[pl.ds(off, CHUNK)])
            pltpu.sync_copy(zrow_v, degi_sh.at[pl.ds(off, CHUNK)])
        pltpu.sync_copy(edges_hbm.at[0, wid], si_v)
        pltpu.sync_copy(edges_hbm.at[1, wid], di_v)
        plsc.subcore_barrier()

        def body(j, carry):
            pltpu.sync_copy(ones_v, dego_sh.at[si_v.at[j]], add=True)
            pltpu.sync_copy(ones_v, degi_sh.at[di_v.at[j]], add=True)
            return carry

        lax.fori_loop(0, nchunk, body, 0)
        plsc.subcore_barrier()
        for k2 in range(ROWS_PT // CHUNK):
            off = s * ROWS_PT + k2 * CHUNK
            pltpu.sync_copy(dego_sh.at[pl.ds(off, CHUNK)],
                            out_hbm.at[c, 0, pl.ds(off, CHUNK)])
            pltpu.sync_copy(degi_sh.at[pl.ds(off, CHUNK)],
                            out_hbm.at[c, 1, pl.ds(off, CHUNK)])

    return k(edges4)


def _sc_propagate(hs, edges4):
    nchunk = edges4.shape[2]

    npass = 2
    assert nchunk % (2 * npass) == 0
    cpp = nchunk // npass

    @functools.partial(
        pl.kernel,
        out_type=jax.ShapeDtypeStruct((NC, NPAD, D), jnp.float32),
        mesh=_mesh(),
        scratch_types=[
            pltpu.VMEM((cpp, CHUNK), jnp.int32),
            pltpu.VMEM((cpp, CHUNK), jnp.int32),
            pltpu.VMEM((CHUNK, D), jnp.float32),
            pltpu.VMEM((CHUNK, D), jnp.float32),
            pltpu.VMEM_SHARED((NPAD, D), jnp.float32),
            pltpu.SemaphoreType.DMA,
            pltpu.SemaphoreType.DMA,
        ],
    )
    def k(hs_hbm, edges_hbm, out_hbm, si_v, di_v, rows0, rows1, acc_sh,
          sem0, sem1):
        c = lax.axis_index("c")
        s = lax.axis_index("s")
        wid = s * NC + c

        def zbody(j, carry):
            for i in range(D // 16):
                rows0[j, pl.ds(i * 16, 16)] = jnp.zeros((16,), jnp.float32)
            return carry

        lax.fori_loop(0, CHUNK, zbody, 0)
        for k2 in range(ROWS_PT // CHUNK):
            off = s * ROWS_PT + k2 * CHUNK
            pltpu.sync_copy(rows0, acc_sh.at[pl.ds(off, CHUNK)])
        plsc.subcore_barrier()

        for p in range(npass):
            pltpu.sync_copy(edges_hbm.at[0, wid, pl.ds(p * cpp, cpp)], si_v)
            pltpu.sync_copy(edges_hbm.at[1, wid, pl.ds(p * cpp, cpp)], di_v)
            pltpu.async_copy(hs_hbm.at[si_v.at[0]], rows0, sem0)

            def body(j2, carry):
                j = 2 * j2
                pltpu.make_async_copy(hs_hbm.at[si_v.at[j]], rows0, sem0).wait()
                pltpu.async_copy(hs_hbm.at[si_v.at[j + 1]], rows1, sem1)
                pltpu.sync_copy(rows0, acc_sh.at[di_v.at[j]], add=True)
                pltpu.make_async_copy(
                    hs_hbm.at[si_v.at[j + 1]], rows1, sem1).wait()
                pltpu.async_copy(hs_hbm.at[si_v.at[j + 2]], rows0, sem0)
                pltpu.sync_copy(rows1, acc_sh.at[di_v.at[j + 1]], add=True)
                return carry

            lax.fori_loop(0, (cpp - 2) // 2, body, 0)
            pltpu.make_async_copy(hs_hbm.at[si_v.at[cpp - 2]], rows0,
                                  sem0).wait()
            pltpu.async_copy(hs_hbm.at[si_v.at[cpp - 1]], rows1, sem1)
            pltpu.sync_copy(rows0, acc_sh.at[di_v.at[cpp - 2]], add=True)
            pltpu.make_async_copy(hs_hbm.at[si_v.at[cpp - 1]], rows1,
                                  sem1).wait()
            pltpu.sync_copy(rows1, acc_sh.at[di_v.at[cpp - 1]], add=True)
        plsc.subcore_barrier()
        for k2 in range(ROWS_PT // CHUNK):
            off = s * ROWS_PT + k2 * CHUNK
            pltpu.sync_copy(acc_sh.at[pl.ds(off, CHUNK)],
                            out_hbm.at[c, pl.ds(off, CHUNK)])

    return k(hs, edges4)


def _tc_sfactor(degp):
    def body(degp_ref, sc_ref):
        dego = degp_ref[0, 0, :] + degp_ref[1, 0, :]
        degi = degp_ref[0, 1, :] + degp_ref[1, 1, :]
        sc_ref[0, :] = lax.rsqrt(jnp.maximum(dego, 1.0))
        sc_ref[1, :] = lax.rsqrt(jnp.maximum(degi, 1.0))

    return pl.pallas_call(
        body, out_shape=jax.ShapeDtypeStruct((2, NPAD), jnp.float32))(degp)


BM = 1280


def _tc_mm0(h, W):
    def body(h_ref, w_ref, o_ref):
        o_ref[...] = jnp.dot(h_ref[...], w_ref[...],
                             preferred_element_type=jnp.float32)

    return pl.pallas_call(
        body,
        grid=(NPAD // BM,),
        in_specs=[
            pl.BlockSpec((BM, D), lambda i: (i, 0)),
            pl.BlockSpec((D, D), lambda i: (0, 0)),
        ],
        out_specs=pl.BlockSpec((BM, D), lambda i: (i, 0)),
        out_shape=jax.ShapeDtypeStruct((NPAD, D), jnp.float32),
    )(h, W)


def _tc_scale0(h, scn):
    def body(h_ref, scn_ref, o_ref):
        o_ref[...] = h_ref[...] * scn_ref[:, 0:1]

    return pl.pallas_call(
        body,
        grid=(NPAD // BM,),
        in_specs=[
            pl.BlockSpec((BM, D), lambda i: (i, 0)),
            pl.BlockSpec((BM, 2), lambda i: (i, 0)),
        ],
        out_specs=pl.BlockSpec((BM, D), lambda i: (i, 0)),
        out_shape=jax.ShapeDtypeStruct((NPAD, D), jnp.float32),
    )(h, scn)


def _tc_mid(P, scn, b0r, W1):
    def body(p_ref, scn_ref, b_ref, w_ref, o_ref):
        agg = p_ref[0] + p_ref[1]
        t = jnp.maximum(agg * scn_ref[:, 1:2] + b_ref[...], 0.0)
        o_ref[...] = jnp.dot(
            t, w_ref[...], preferred_element_type=jnp.float32) * scn_ref[:, 0:1]

    return pl.pallas_call(
        body,
        grid=(NPAD // BM,),
        in_specs=[
            pl.BlockSpec((NC, BM, D), lambda i: (0, i, 0)),
            pl.BlockSpec((BM, 2), lambda i: (i, 0)),
            pl.BlockSpec((1, D), lambda i: (0, 0)),
            pl.BlockSpec((D, D), lambda i: (0, 0)),
        ],
        out_specs=pl.BlockSpec((BM, D), lambda i: (i, 0)),
        out_shape=jax.ShapeDtypeStruct((NPAD, D), jnp.float32),
    )(P, scn, b0r, W1)


BMF = 1000


def _tc_final(P, scn, b1r):
    def body(p_ref, scn_ref, b_ref, o_ref):
        o_ref[...] = (p_ref[0] + p_ref[1]) * scn_ref[:, 1:2] + b_ref[...]

    return pl.pallas_call(
        body,
        grid=(N_NODES // BMF,),
        in_specs=[
            pl.BlockSpec((NC, BMF, D), lambda i: (0, i, 0)),
            pl.BlockSpec((BMF, 2), lambda i: (i, 0)),
            pl.BlockSpec((1, D), lambda i: (0, 0)),
        ],
        out_specs=pl.BlockSpec((BMF, D), lambda i: (i, 0)),
        out_shape=jax.ShapeDtypeStruct((N_NODES, D), jnp.float32),
    )(P, scn, b1r)


def kernel(x, edge_index, W0, b0, W1, b1):
    n, E = x.shape[0], edge_index.shape[1]
    nchunk = -(-E // (NW * CHUNK))
    nchunk += nchunk % 2
    epad = NW * nchunk * CHUNK - E
    fill = N_NODES + jnp.arange(epad, dtype=jnp.int32) % (NPAD - N_NODES)
    fill2 = jnp.broadcast_to(fill, (2, epad))
    edges4 = jnp.concatenate([edge_index, fill2], axis=1).reshape(
        2, NW, nchunk, CHUNK)
    xp = jnp.pad(x, ((0, NPAD - n), (0, 0)))

    degp = _sc_degrees(edges4)
    h0 = _tc_mm0(xp, W0)
    scn = _tc_sfactor(degp).T
    b0r = b0.reshape(1, D)
    b1r = b1.reshape(1, D)

    hs0 = _tc_scale0(h0, scn)
    P0 = _sc_propagate(hs0, edges4)
    hs1 = _tc_mid(P0, scn, b0r, W1)
    P1 = _sc_propagate(hs1, edges4)
    return _tc_final(P1, scn, b1r)


# speedup vs baseline: 3.3368x; 1.0296x over previous
import functools

import jax
import jax.numpy as jnp
from jax import lax
from jax.experimental import pallas as pl
from jax.experimental.pallas import tpu as pltpu
from jax.experimental.pallas import tpu_sc as plsc

N_NODES = 10000
D = 128
NPAD = 10240
NC = 2
NS = 16
NW = NC * NS
CHUNK = 128
ROWS_PT = NPAD // NS

_mesh = lambda: plsc.VectorSubcoreMesh(core_axis_name="c", subcore_axis_name="s")


def _sc_degrees(edges4):
    nchunk = edges4.shape[2]

    @functools.partial(
        pl.kernel,
        out_type=jax.ShapeDtypeStruct((NC, 2, NPAD), jnp.float32),
        mesh=_mesh(),
        scratch_types=[
            pltpu.VMEM((nchunk, CHUNK), jnp.int32),
            pltpu.VMEM((nchunk, CHUNK), jnp.int32),
            pltpu.VMEM((CHUNK,), jnp.float32),
            pltpu.VMEM((CHUNK,), jnp.float32),
            pltpu.VMEM_SHARED((NPAD,), jnp.float32),
            pltpu.VMEM_SHARED((NPAD,), jnp.float32),
            pltpu.SemaphoreType.DMA,
        ],
    )
    def k(edges_hbm, out_hbm, si_v, di_v, ones_v, zrow_v, dego_sh, degi_sh,
          sem):
        c = lax.axis_index("c")
        s = lax.axis_index("s")
        wid = s * NC + c
        for i in range(CHUNK // 16):
            ones_v[pl.ds(i * 16, 16)] = jnp.ones((16,), jnp.float32)
            zrow_v[pl.ds(i * 16, 16)] = jnp.zeros((16,), jnp.float32)
        for k2 in range(ROWS_PT // CHUNK):
            off = s * ROWS_PT + k2 * CHUNK
            pltpu.sync_copy(zrow_v, dego_sh.at[pl.ds(off, CHUNK)])
            pltpu.sync_copy(zrow_v, degi_sh.at[pl.ds(off, CHUNK)])
        pltpu.sync_copy(edges_hbm.at[0, wid], si_v)
        pltpu.sync_copy(edges_hbm.at[1, wid], di_v)
        plsc.subcore_barrier()

        def body(j, carry):
            pltpu.async_copy(ones_v, dego_sh.at[si_v.at[j]], sem, add=True)
            pltpu.async_copy(ones_v, degi_sh.at[di_v.at[j]], sem, add=True)
            return carry

        lax.fori_loop(0, nchunk, body, 0)

        def dbody(j, carry):
            pltpu.make_async_copy(ones_v, dego_sh.at[si_v.at[j]], sem).wait()
            pltpu.make_async_copy(ones_v, degi_sh.at[di_v.at[j]], sem).wait()
            return carry

        lax.fori_loop(0, nchunk, dbody, 0)
        plsc.subcore_barrier()
        for k2 in range(ROWS_PT // CHUNK):
            off = s * ROWS_PT + k2 * CHUNK
            pltpu.sync_copy(dego_sh.at[pl.ds(off, CHUNK)],
                            out_hbm.at[c, 0, pl.ds(off, CHUNK)])
            pltpu.sync_copy(degi_sh.at[pl.ds(off, CHUNK)],
                            out_hbm.at[c, 1, pl.ds(off, CHUNK)])

    return k(edges4)


def _sc_propagate(hs, edges4):
    nchunk = edges4.shape[2]

    npass = 2
    assert nchunk % (2 * npass) == 0
    cpp = nchunk // npass

    @functools.partial(
        pl.kernel,
        out_type=jax.ShapeDtypeStruct((NC, NPAD, D), jnp.float32),
        mesh=_mesh(),
        scratch_types=[
            pltpu.VMEM((cpp, CHUNK), jnp.int32),
            pltpu.VMEM((cpp, CHUNK), jnp.int32),
            pltpu.VMEM((CHUNK, D), jnp.float32),
            pltpu.VMEM((CHUNK, D), jnp.float32),
            pltpu.VMEM_SHARED((NPAD, D), jnp.float32),
            pltpu.SemaphoreType.DMA,
            pltpu.SemaphoreType.DMA,
        ],
    )
    def k(hs_hbm, edges_hbm, out_hbm, si_v, di_v, rows0, rows1, acc_sh,
          sem0, sem1):
        c = lax.axis_index("c")
        s = lax.axis_index("s")
        wid = s * NC + c

        def zbody(j, carry):
            for i in range(D // 16):
                rows0[j, pl.ds(i * 16, 16)] = jnp.zeros((16,), jnp.float32)
            return carry

        lax.fori_loop(0, CHUNK, zbody, 0)
        for k2 in range(ROWS_PT // CHUNK):
            off = s * ROWS_PT + k2 * CHUNK
            pltpu.sync_copy(rows0, acc_sh.at[pl.ds(off, CHUNK)])
        plsc.subcore_barrier()

        for p in range(npass):
            pltpu.sync_copy(edges_hbm.at[0, wid, pl.ds(p * cpp, cpp)], si_v)
            pltpu.sync_copy(edges_hbm.at[1, wid, pl.ds(p * cpp, cpp)], di_v)
            pltpu.async_copy(hs_hbm.at[si_v.at[0]], rows0, sem0)

            def body(j2, carry):
                j = 2 * j2
                pltpu.make_async_copy(hs_hbm.at[si_v.at[j]], rows0, sem0).wait()
                pltpu.async_copy(hs_hbm.at[si_v.at[j + 1]], rows1, sem1)
                pltpu.sync_copy(rows0, acc_sh.at[di_v.at[j]], add=True)
                pltpu.make_async_copy(
                    hs_hbm.at[si_v.at[j + 1]], rows1, sem1).wait()
                pltpu.async_copy(hs_hbm.at[si_v.at[j + 2]], rows0, sem0)
                pltpu.sync_copy(rows1, acc_sh.at[di_v.at[j + 1]], add=True)
                return carry

            lax.fori_loop(0, (cpp - 2) // 2, body, 0)
            pltpu.make_async_copy(hs_hbm.at[si_v.at[cpp - 2]], rows0,
                                  sem0).wait()
            pltpu.async_copy(hs_hbm.at[si_v.at[cpp - 1]], rows1, sem1)
            pltpu.sync_copy(rows0, acc_sh.at[di_v.at[cpp - 2]], add=True)
            pltpu.make_async_copy(hs_hbm.at[si_v.at[cpp - 1]], rows1,
                                  sem1).wait()
            pltpu.sync_copy(rows1, acc_sh.at[di_v.at[cpp - 1]], add=True)
        plsc.subcore_barrier()
        for k2 in range(ROWS_PT // CHUNK):
            off = s * ROWS_PT + k2 * CHUNK
            pltpu.sync_copy(acc_sh.at[pl.ds(off, CHUNK)],
                            out_hbm.at[c, pl.ds(off, CHUNK)])

    return k(hs, edges4)


def _tc_sfactor(degp):
    def body(degp_ref, sc_ref):
        dego = degp_ref[0, 0, :] + degp_ref[1, 0, :]
        degi = degp_ref[0, 1, :] + degp_ref[1, 1, :]
        sc_ref[0, :] = lax.rsqrt(jnp.maximum(dego, 1.0))
        sc_ref[1, :] = lax.rsqrt(jnp.maximum(degi, 1.0))

    return pl.pallas_call(
        body, out_shape=jax.ShapeDtypeStruct((2, NPAD), jnp.float32))(degp)


BM = 1280


def _tc_mm0(h, W):
    def body(h_ref, w_ref, o_ref):
        o_ref[...] = jnp.dot(h_ref[...], w_ref[...],
                             preferred_element_type=jnp.float32)

    return pl.pallas_call(
        body,
        grid=(NPAD // BM,),
        in_specs=[
            pl.BlockSpec((BM, D), lambda i: (i, 0)),
            pl.BlockSpec((D, D), lambda i: (0, 0)),
        ],
        out_specs=pl.BlockSpec((BM, D), lambda i: (i, 0)),
        out_shape=jax.ShapeDtypeStruct((NPAD, D), jnp.float32),
    )(h, W)


def _tc_scale0(h, scn):
    def body(h_ref, scn_ref, o_ref):
        o_ref[...] = h_ref[...] * scn_ref[:, 0:1]

    return pl.pallas_call(
        body,
        grid=(NPAD // BM,),
        in_specs=[
            pl.BlockSpec((BM, D), lambda i: (i, 0)),
            pl.BlockSpec((BM, 2), lambda i: (i, 0)),
        ],
        out_specs=pl.BlockSpec((BM, D), lambda i: (i, 0)),
        out_shape=jax.ShapeDtypeStruct((NPAD, D), jnp.float32),
    )(h, scn)


def _tc_mid(P, scn, b0r, W1):
    def body(p_ref, scn_ref, b_ref, w_ref, o_ref):
        agg = p_ref[0] + p_ref[1]
        t = jnp.maximum(agg * scn_ref[:, 1:2] + b_ref[...], 0.0)
        o_ref[...] = jnp.dot(
            t, w_ref[...], preferred_element_type=jnp.float32) * scn_ref[:, 0:1]

    return pl.pallas_call(
        body,
        grid=(NPAD // BM,),
        in_specs=[
            pl.BlockSpec((NC, BM, D), lambda i: (0, i, 0)),
            pl.BlockSpec((BM, 2), lambda i: (i, 0)),
            pl.BlockSpec((1, D), lambda i: (0, 0)),
            pl.BlockSpec((D, D), lambda i: (0, 0)),
        ],
        out_specs=pl.BlockSpec((BM, D), lambda i: (i, 0)),
        out_shape=jax.ShapeDtypeStruct((NPAD, D), jnp.float32),
    )(P, scn, b0r, W1)


BMF = 1000


def _tc_final(P, scn, b1r):
    def body(p_ref, scn_ref, b_ref, o_ref):
        o_ref[...] = (p_ref[0] + p_ref[1]) * scn_ref[:, 1:2] + b_ref[...]

    return pl.pallas_call(
        body,
        grid=(N_NODES // BMF,),
        in_specs=[
            pl.BlockSpec((NC, BMF, D), lambda i: (0, i, 0)),
            pl.BlockSpec((BMF, 2), lambda i: (i, 0)),
            pl.BlockSpec((1, D), lambda i: (0, 0)),
        ],
        out_specs=pl.BlockSpec((BMF, D), lambda i: (i, 0)),
        out_shape=jax.ShapeDtypeStruct((N_NODES, D), jnp.float32),
    )(P, scn, b1r)


def kernel(x, edge_index, W0, b0, W1, b1):
    n, E = x.shape[0], edge_index.shape[1]
    nchunk = -(-E // (NW * CHUNK))
    nchunk += nchunk % 2
    epad = NW * nchunk * CHUNK - E
    fill = N_NODES + jnp.arange(epad, dtype=jnp.int32) % (NPAD - N_NODES)
    fill2 = jnp.broadcast_to(fill, (2, epad))
    edges4 = jnp.concatenate([edge_index, fill2], axis=1).reshape(
        2, NW, nchunk, CHUNK)
    xp = jnp.pad(x, ((0, NPAD - n), (0, 0)))

    degp = _sc_degrees(edges4)
    h0 = _tc_mm0(xp, W0)
    scn = _tc_sfactor(degp).T
    b0r = b0.reshape(1, D)
    b1r = b1.reshape(1, D)

    hs0 = _tc_scale0(h0, scn)
    P0 = _sc_propagate(hs0, edges4)
    hs1 = _tc_mid(P0, scn, b0r, W1)
    P1 = _sc_propagate(hs1, edges4)
    return _tc_final(P1, scn, b1r)
